# baseline (speedup 1.0000x reference)
import functools

import jax
import jax.numpy as jnp
from jax import lax
from jax.experimental import pallas as pl
from jax.experimental.pallas import tpu as pltpu
from jax.experimental.pallas import tpu_sc as plsc

V = 1000000
D = 64
B = 32
N = 32
K = 100
P = B * N
W = 112
VP = V // 2
NORM_TERM = 13.815510557964274
MIN_PROB = 1e-9

NC = 2
NS = 16
NW = NC * NS
PB = P // NW
KB = W // 16
RS = 4


def _sc_kernel_body(idx_hbm, inp_hbm, embp_hbm, bias_hbm, noise_hbm,
                    s_out, q_out, b_out,
                    idx_v, pair_v, h_v, rows_v, q_v, b_v, s_v,
                    sem_r, sem_q, sem_b):
    wid = lax.axis_index("s") * NC + lax.axis_index("c")
    base = wid * PB
    pltpu.sync_copy(idx_hbm.at[pl.ds(base, PB)], idx_v)
    pltpu.sync_copy(inp_hbm.at[pl.ds(base, PB)], h_v)

    lane = lax.iota(jnp.int32, 16)

    for p in range(PB):
        for kb in range(KB):
            pair_v[p, pl.ds(kb * 16, 16)] = (
                jax.lax.shift_right_logical(idx_v[p, pl.ds(kb * 16, 16)], 1))

    for p in range(PB):
        pltpu.async_copy(noise_hbm.at[idx_v.at[p]], q_v.at[p], sem_q)
        pltpu.async_copy(bias_hbm.at[idx_v.at[p]], b_v.at[p], sem_b)

    def issue_rows(p, slot):
        pltpu.async_copy(embp_hbm.at[pair_v.at[p]], rows_v.at[slot],
                         sem_r.at[slot])

    def wait_rows(slot):
        pltpu.make_async_copy(embp_hbm.at[pl.ds(0, W)], rows_v.at[slot],
                              sem_r.at[slot]).wait()

    for p in range(RS - 1):
        issue_rows(p, p % RS)

    def pos_body(p, carry):
        slot = lax.rem(p, RS)

        @pl.when(p + RS - 1 < PB)
        def _():
            issue_rows(p + RS - 1, lax.rem(p + RS - 1, RS))

        wait_rows(slot)

        hs = [h_v[p, pl.ds(j * 16, 16)] for j in range(D // 16)]
        for kb in range(KB):
            ivec = idx_v[p, pl.ds(kb * 16, 16)]
            svec = jnp.zeros((16,), jnp.float32)
            for j in range(16):
                k = kb * 16 + j
                off = (ivec[j] & 1) * D
                acc = rows_v[slot, k, pl.ds(off, 16)] * hs[0]
                for t in range(1, D // 16):
                    acc = acc + rows_v[slot, k, pl.ds(off + t * 16, 16)] * hs[t]
                svec = jnp.where(lane == j, jnp.sum(acc), svec)
            s_v[p, pl.ds(kb * 16, 16)] = svec
        return carry

    lax.fori_loop(0, PB, pos_body, 0)

    for p in range(PB):
        pltpu.make_async_copy(noise_hbm.at[pl.ds(0, W)], q_v.at[p],
                              sem_q).wait()
        pltpu.make_async_copy(bias_hbm.at[pl.ds(0, W)], b_v.at[p],
                              sem_b).wait()
    pltpu.sync_copy(s_v, s_out.at[pl.ds(base, PB)])
    pltpu.sync_copy(q_v, q_out.at[pl.ds(base, PB)])
    pltpu.sync_copy(b_v, b_out.at[pl.ds(base, PB)])


def _tc_epilogue_body(s_ref, q_ref, b_ref, o_ref):
    s = s_ref[...] + b_ref[...]
    q = q_ref[...]
    p = jnp.clip(jnp.exp(s - NORM_TERM), MIN_PROB, 1.0)
    pt = p / (p + float(K) * q)
    col = lax.broadcasted_iota(jnp.int32, s.shape, 1)
    logp = jnp.maximum(jnp.log(pt), -100.0)
    log1mp = jnp.maximum(jnp.log(1.0 - pt), -100.0)
    bce = jnp.where(col == 0, -logp, -log1mp)
    bce = jnp.where(col < K + 1, bce, 0.0)
    o_ref[...] = (jnp.sum(bce) * (1.0 / P)).reshape(1, 1)


def kernel(target, inp, noise_samples, noise, emb_weight, emb_bias):
    idx = jnp.concatenate(
        [target.reshape(P, 1), noise_samples.reshape(P, K)], axis=1)
    idx = jnp.concatenate(
        [idx, jnp.zeros((P, W - (K + 1)), jnp.int32)], axis=1).astype(jnp.int32)
    inp2d = inp.reshape(P, D).astype(jnp.float32)
    embp = emb_weight.reshape(VP, 2 * D)

    mesh = plsc.VectorSubcoreMesh(core_axis_name="c", subcore_axis_name="s")
    sc = pl.kernel(
        _sc_kernel_body,
        mesh=mesh,
        compiler_params=pltpu.CompilerParams(
            needs_layout_passes=False, use_tc_tiling_on_sc=True),
        out_type=[
            jax.ShapeDtypeStruct((P, W), jnp.float32),
            jax.ShapeDtypeStruct((P, W), jnp.float32),
            jax.ShapeDtypeStruct((P, W), jnp.float32),
        ],
        scratch_types=[
            pltpu.VMEM((PB, W), jnp.int32),
            pltpu.VMEM((PB, W), jnp.int32),
            pltpu.VMEM((PB, D), jnp.float32),
            pltpu.VMEM((RS, W, 2 * D), jnp.float32),
            pltpu.VMEM((PB, W), jnp.float32),
            pltpu.VMEM((PB, W), jnp.float32),
            pltpu.VMEM((PB, W), jnp.float32),
            pltpu.SemaphoreType.DMA((RS,)),
            pltpu.SemaphoreType.DMA,
            pltpu.SemaphoreType.DMA,
        ],
    )
    scores, qvals, bvals = sc(idx, inp2d, embp, emb_bias, noise)

    out = pl.pallas_call(
        _tc_epilogue_body,
        out_shape=jax.ShapeDtypeStruct((1, 1), jnp.float32),
    )(scores, qvals, bvals)
    return out[0, 0]


# speedup vs baseline: 1.0141x; 1.0141x over previous
import functools

import jax
import jax.numpy as jnp
from jax import lax
from jax.experimental import pallas as pl
from jax.experimental.pallas import tpu as pltpu
from jax.experimental.pallas import tpu_sc as plsc

V = 1000000
D = 64
B = 32
N = 32
K = 100
P = B * N
W = 112
VP = V // 2
NORM_TERM = 13.815510557964274
MIN_PROB = 1e-9

NC = 2
NS = 16
NW = NC * NS
PB = P // NW
KB = W // 16
RS = 4
CHUNKS = ((0, 32), (32, 32), (64, 32), (96, 16))

DEPAD_BK = 5000


def _depad_body(a_ref, b_ref, o_ref):
    o_ref[...] = jnp.concatenate([a_ref[...], b_ref[...]], axis=1)


def _sc_kernel_body(idx_hbm, inp_hbm, embp_hbm, bias_hbm, noise_hbm,
                    s_out, q_out, b_out,
                    idx_v, pair_v, h_v, rows_v, q_v, b_v, s_v,
                    sem_r, sem_q, sem_b):
    wid = lax.axis_index("s") * NC + lax.axis_index("c")
    base = wid * PB
    pltpu.sync_copy(idx_hbm.at[pl.ds(base, PB)], idx_v)
    pltpu.sync_copy(inp_hbm.at[pl.ds(base, PB)], h_v)

    lane = lax.iota(jnp.int32, 16)

    for p in range(PB):
        for kb in range(KB):
            iv = idx_v[p, pl.ds(kb * 16, 16)]
            pair_v[p, pl.ds(kb * 16, 16)] = jnp.where(
                iv >= VP, iv - VP, iv)

    for p in range(PB):
        pltpu.async_copy(noise_hbm.at[idx_v.at[p]], q_v.at[p], sem_q)
        pltpu.async_copy(bias_hbm.at[idx_v.at[p]], b_v.at[p], sem_b)

    def issue_rows(p, slot):
        for off, sz in CHUNKS:
            pltpu.async_copy(
                embp_hbm.at[pair_v.at[p, pl.ds(off, sz)]],
                rows_v.at[slot, pl.ds(off, sz)],
                sem_r.at[slot])

    def wait_rows(slot):
        for off, sz in CHUNKS:
            pltpu.make_async_copy(
                embp_hbm.at[pl.ds(0, sz)],
                rows_v.at[slot, pl.ds(off, sz)],
                sem_r.at[slot]).wait()

    for p in range(RS - 1):
        issue_rows(p, p % RS)

    def pos_body(p, carry):
        slot = lax.rem(p, RS)

        @pl.when(p + RS - 1 < PB)
        def _():
            issue_rows(p + RS - 1, lax.rem(p + RS - 1, RS))

        wait_rows(slot)

        hs = [h_v[p, pl.ds(j * 16, 16)] for j in range(D // 16)]
        for kb in range(KB):
            ivec = idx_v[p, pl.ds(kb * 16, 16)]
            svec = jnp.zeros((16,), jnp.float32)
            for j in range(16):
                k = kb * 16 + j
                off = jnp.where(ivec[j] >= VP, D, 0)
                acc = rows_v[slot, k, pl.ds(off, 16)] * hs[0]
                for t in range(1, D // 16):
                    acc = acc + rows_v[slot, k, pl.ds(off + t * 16, 16)] * hs[t]
                svec = jnp.where(lane == j, jnp.sum(acc), svec)
            s_v[p, pl.ds(kb * 16, 16)] = svec
        return carry

    lax.fori_loop(0, PB, pos_body, 0)

    for p in range(PB):
        pltpu.make_async_copy(noise_hbm.at[pl.ds(0, W)], q_v.at[p],
                              sem_q).wait()
        pltpu.make_async_copy(bias_hbm.at[pl.ds(0, W)], b_v.at[p],
                              sem_b).wait()
    pltpu.sync_copy(s_v, s_out.at[pl.ds(base, PB)])
    pltpu.sync_copy(q_v, q_out.at[pl.ds(base, PB)])
    pltpu.sync_copy(b_v, b_out.at[pl.ds(base, PB)])


def _tc_epilogue_body(s_ref, q_ref, b_ref, o_ref):
    s = s_ref[...] + b_ref[...]
    q = q_ref[...]
    p = jnp.clip(jnp.exp(s - NORM_TERM), MIN_PROB, 1.0)
    pt = p / (p + float(K) * q)
    col = lax.broadcasted_iota(jnp.int32, s.shape, 1)
    logp = jnp.maximum(jnp.log(pt), -100.0)
    log1mp = jnp.maximum(jnp.log(1.0 - pt), -100.0)
    bce = jnp.where(col == 0, -logp, -log1mp)
    bce = jnp.where(col < K + 1, bce, 0.0)
    o_ref[...] = (jnp.sum(bce) * (1.0 / P)).reshape(1, 1)


def kernel(target, inp, noise_samples, noise, emb_weight, emb_bias):
    idx = jnp.concatenate(
        [target.reshape(P, 1), noise_samples.reshape(P, K)], axis=1)
    idx = jnp.concatenate(
        [idx, jnp.zeros((P, W - (K + 1)), jnp.int32)], axis=1).astype(jnp.int32)
    inp2d = inp.reshape(P, D).astype(jnp.float32)

    nblk = VP // DEPAD_BK
    embp = pl.pallas_call(
        _depad_body,
        grid=(nblk,),
        in_specs=[
            pl.BlockSpec((DEPAD_BK, D), lambda i: (i, 0)),
            pl.BlockSpec((DEPAD_BK, D), lambda i: (i + nblk, 0)),
        ],
        out_specs=pl.BlockSpec((DEPAD_BK, 2 * D), lambda i: (i, 0)),
        out_shape=jax.ShapeDtypeStruct((VP, 2 * D), jnp.float32),
    )(emb_weight, emb_weight)

    mesh = plsc.VectorSubcoreMesh(core_axis_name="c", subcore_axis_name="s")
    sc = pl.kernel(
        _sc_kernel_body,
        mesh=mesh,
        compiler_params=pltpu.CompilerParams(
            needs_layout_passes=False, use_tc_tiling_on_sc=True),
        out_type=[
            jax.ShapeDtypeStruct((P, W), jnp.float32),
            jax.ShapeDtypeStruct((P, W), jnp.float32),
            jax.ShapeDtypeStruct((P, W), jnp.float32),
        ],
        scratch_types=[
            pltpu.VMEM((PB, W), jnp.int32),
            pltpu.VMEM((PB, W), jnp.int32),
            pltpu.VMEM((PB, D), jnp.float32),
            pltpu.VMEM((RS, W, 2 * D), jnp.float32),
            pltpu.VMEM((PB, W), jnp.float32),
            pltpu.VMEM((PB, W), jnp.float32),
            pltpu.VMEM((PB, W), jnp.float32),
            pltpu.SemaphoreType.DMA((RS,)),
            pltpu.SemaphoreType.DMA,
            pltpu.SemaphoreType.DMA,
        ],
    )
    scores, qvals, bvals = sc(idx, inp2d, embp, emb_bias, noise)

    out = pl.pallas_call(
        _tc_epilogue_body,
        out_shape=jax.ShapeDtypeStruct((1, 1), jnp.float32),
    )(scores, qvals, bvals)
    return out[0, 0]


# speedup vs baseline: 1.2980x; 1.2800x over previous
import functools

import jax
import jax.numpy as jnp
from jax import lax
from jax.experimental import pallas as pl
from jax.experimental.pallas import tpu as pltpu
from jax.experimental.pallas import tpu_sc as plsc

V = 1000000
D = 64
B = 32
N = 32
K = 100
P = B * N
W = 112
NORM_TERM = 13.815510557964274
MIN_PROB = 1e-9

NC = 2
NS = 16
NW = NC * NS
PB = P // NW
KB = W // 16
RS = 4


def _sc_kernel_body(idx_hbm, inp_hbm, emb_hbm, bias_hbm, noise_hbm,
                    s_out, q_out, b_out,
                    idx_v, h_v, rows_v, q_v, b_v, s_v,
                    sem_r, sem_q, sem_b):
    wid = lax.axis_index("s") * NC + lax.axis_index("c")
    base = wid * PB
    pltpu.sync_copy(idx_hbm.at[pl.ds(base, PB)], idx_v)
    pltpu.sync_copy(inp_hbm.at[pl.ds(base, PB)], h_v)

    lane = lax.iota(jnp.int32, 16)

    for p in range(PB):
        pltpu.async_copy(noise_hbm.at[idx_v.at[p]], q_v.at[p], sem_q)
        pltpu.async_copy(bias_hbm.at[idx_v.at[p]], b_v.at[p], sem_b)

    def issue_rows(p, slot):
        for kb in range(KB):
            ivec = idx_v[p, pl.ds(kb * 16, 16)]
            for j in range(16):
                pltpu.async_copy(
                    emb_hbm.at[ivec[j]],
                    rows_v.at[slot, kb * 16 + j],
                    sem_r.at[slot])

    def wait_rows(slot):
        for _ in range(W):
            pltpu.make_async_copy(
                emb_hbm.at[0],
                rows_v.at[slot, 0],
                sem_r.at[slot]).wait()

    for p in range(RS - 1):
        issue_rows(p, p % RS)

    def pos_body(p, carry):
        slot = lax.rem(p, RS)

        @pl.when(p + RS - 1 < PB)
        def _():
            issue_rows(p + RS - 1, lax.rem(p + RS - 1, RS))

        wait_rows(slot)

        hs = [h_v[p, pl.ds(j * 16, 16)] for j in range(D // 16)]
        for kb in range(KB):
            svec = jnp.zeros((16,), jnp.float32)
            for j in range(16):
                k = kb * 16 + j
                acc = rows_v[slot, k, pl.ds(0, 16)] * hs[0]
                for t in range(1, D // 16):
                    acc = acc + rows_v[slot, k, pl.ds(t * 16, 16)] * hs[t]
                svec = jnp.where(lane == j, jnp.sum(acc), svec)
            s_v[p, pl.ds(kb * 16, 16)] = svec
        return carry

    lax.fori_loop(0, PB, pos_body, 0)

    for p in range(PB):
        pltpu.make_async_copy(noise_hbm.at[pl.ds(0, W)], q_v.at[p],
                              sem_q).wait()
        pltpu.make_async_copy(bias_hbm.at[pl.ds(0, W)], b_v.at[p],
                              sem_b).wait()
    pltpu.sync_copy(s_v, s_out.at[pl.ds(base, PB)])
    pltpu.sync_copy(q_v, q_out.at[pl.ds(base, PB)])
    pltpu.sync_copy(b_v, b_out.at[pl.ds(base, PB)])


def _tc_epilogue_body(s_ref, q_ref, b_ref, o_ref):
    s = s_ref[...] + b_ref[...]
    q = q_ref[...]
    p = jnp.clip(jnp.exp(s - NORM_TERM), MIN_PROB, 1.0)
    pt = p / (p + float(K) * q)
    col = lax.broadcasted_iota(jnp.int32, s.shape, 1)
    logp = jnp.maximum(jnp.log(pt), -100.0)
    log1mp = jnp.maximum(jnp.log(1.0 - pt), -100.0)
    bce = jnp.where(col == 0, -logp, -log1mp)
    bce = jnp.where(col < K + 1, bce, 0.0)
    o_ref[...] = (jnp.sum(bce) * (1.0 / P)).reshape(1, 1)


def kernel(target, inp, noise_samples, noise, emb_weight, emb_bias):
    idx = jnp.concatenate(
        [target.reshape(P, 1), noise_samples.reshape(P, K)], axis=1)
    idx = jnp.concatenate(
        [idx, jnp.zeros((P, W - (K + 1)), jnp.int32)], axis=1).astype(jnp.int32)
    inp2d = inp.reshape(P, D).astype(jnp.float32)

    mesh = plsc.VectorSubcoreMesh(core_axis_name="c", subcore_axis_name="s")
    sc = pl.kernel(
        _sc_kernel_body,
        mesh=mesh,
        compiler_params=pltpu.CompilerParams(
            needs_layout_passes=False, use_tc_tiling_on_sc=True),
        out_type=[
            jax.ShapeDtypeStruct((P, W), jnp.float32),
            jax.ShapeDtypeStruct((P, W), jnp.float32),
            jax.ShapeDtypeStruct((P, W), jnp.float32),
        ],
        scratch_types=[
            pltpu.VMEM((PB, W), jnp.int32),
            pltpu.VMEM((PB, D), jnp.float32),
            pltpu.VMEM((RS, W, D), jnp.float32),
            pltpu.VMEM((PB, W), jnp.float32),
            pltpu.VMEM((PB, W), jnp.float32),
            pltpu.VMEM((PB, W), jnp.float32),
            pltpu.SemaphoreType.DMA((RS,)),
            pltpu.SemaphoreType.DMA,
            pltpu.SemaphoreType.DMA,
        ],
    )
    scores, qvals, bvals = sc(idx, inp2d, emb_weight, emb_bias, noise)

    out = pl.pallas_call(
        _tc_epilogue_body,
        out_shape=jax.ShapeDtypeStruct((1, 1), jnp.float32),
    )(scores, qvals, bvals)
    return out[0, 0]
